# Initial kernel scaffold; baseline (speedup 1.0000x reference)
import jax
import jax.numpy as jnp
from jax.experimental import pallas as pl


def kernel(x, W, b, emb):
    raise NotImplementedError("write your pallas kernel here")


# speedup vs baseline: 1.3198x; 1.3198x over previous
import functools

import jax
import jax.numpy as jnp
from jax import lax
from jax.experimental import pallas as pl
from jax.experimental.pallas import tpu as pltpu
from jax.experimental.pallas import tpu_sc as plsc

N, C_IN, H, W_SP = 16, 96, 32, 32
HW = H * W_SP
K, D = 8192, 32
KB = 1024
KBC = K // KB
_LOSS_SCALE = 1.25 / float(N * D * HW)


def _vq_tc_body(x_ref, w_ref, b_ref, emb_ref, idx_ref, loss_ref,
                z_scr, z2_scr, minv_scr, arg_scr, acc_scr):
    n = pl.program_id(0)
    k = pl.program_id(1)

    @pl.when(k == 0)
    def _init():
        z = lax.dot_general(w_ref[...].astype(jnp.bfloat16),
                            x_ref[0].astype(jnp.bfloat16),
                            (((1,), (0,)), ((), ())),
                            preferred_element_type=jnp.float32)
        z = z + b_ref[...]
        z_scr[...] = z
        z2_scr[...] = jnp.sum(z * z, axis=0, keepdims=True)
        minv_scr[...] = jnp.full((1, HW), jnp.inf, jnp.float32)
        arg_scr[...] = jnp.zeros((1, HW), jnp.int32)

    @pl.when((n == 0) & (k == 0))
    def _init_acc():
        acc_scr[...] = jnp.zeros((1, 1), jnp.float32)

    emb = emb_ref[...]
    m = lax.dot_general(emb.astype(jnp.bfloat16),
                        z_scr[...].astype(jnp.bfloat16),
                        (((1,), (0,)), ((), ())),
                        preferred_element_type=jnp.float32)
    e2 = jnp.sum(emb * emb, axis=1, keepdims=True)
    score = (z2_scr[...] - 2.0 * m) + e2
    bmin = jnp.min(score, axis=0, keepdims=True)
    rows = lax.broadcasted_iota(jnp.int32, (KB, HW), 0)
    barg = jnp.min(jnp.where(score == bmin, rows, K),
                   axis=0, keepdims=True) + k * KB
    old = minv_scr[...]
    upd = bmin < old
    arg_scr[...] = jnp.where(upd, barg, arg_scr[...])
    newmin = jnp.where(upd, bmin, old)
    rounded = newmin.astype(jnp.bfloat16).astype(jnp.float32)
    minv_scr[...] = jnp.where((k % 4) == 3, rounded, newmin)

    @pl.when(k == KBC - 1)
    def _finish():
        idx_ref[0] = arg_scr[...]
        acc_scr[...] += jnp.sum(minv_scr[...], axis=1, keepdims=True)

        @pl.when(n == N - 1)
        def _emit_loss():
            loss_ref[...] = acc_scr[...] * _LOSS_SCALE


_tc_call = pl.pallas_call(
    _vq_tc_body,
    grid=(N, KBC),
    in_specs=[
        pl.BlockSpec((1, C_IN, HW), lambda n, k: (n, 0, 0)),
        pl.BlockSpec((D, C_IN), lambda n, k: (0, 0)),
        pl.BlockSpec((D, 1), lambda n, k: (0, 0)),
        pl.BlockSpec((KB, D), lambda n, k: (k, 0)),
    ],
    out_specs=[
        pl.BlockSpec((1, 1, HW), lambda n, k: (n, 0, 0)),
        pl.BlockSpec((1, 1), lambda n, k: (0, 0)),
    ],
    out_shape=[
        jax.ShapeDtypeStruct((N, 1, HW), jnp.int32),
        jax.ShapeDtypeStruct((1, 1), jnp.float32),
    ],
    scratch_shapes=[
        pltpu.VMEM((D, HW), jnp.float32),
        pltpu.VMEM((1, HW), jnp.float32),
        pltpu.VMEM((1, HW), jnp.float32),
        pltpu.VMEM((1, HW), jnp.int32),
        pltpu.VMEM((1, 1), jnp.float32),
    ],
)

_NW = 32
_BPW = (N * HW) // _NW
_CH = 128
_NCH = _BPW // _CH


@functools.lru_cache(maxsize=1)
def _sc_gather_call():
    @functools.partial(
        pl.kernel,
        mesh=plsc.VectorSubcoreMesh(core_axis_name="c", subcore_axis_name="s"),
        out_type=jax.ShapeDtypeStruct((N * HW, D), jnp.float32),
        scratch_types=[
            pltpu.VMEM((_NCH, _CH), jnp.int32),
            pltpu.VMEM((_BPW, D), jnp.float32),
            pltpu.SemaphoreType.DMA,
        ],
        compiler_params=pltpu.CompilerParams(use_tc_tiling_on_sc=False),
    )
    def _sc_gather(emb_hbm, idx_hbm, out_hbm, idx_v, rows_v, sem):
        wid = lax.axis_index("s") * 2 + lax.axis_index("c")
        pltpu.sync_copy(idx_hbm.at[wid], idx_v)
        copies = [
            pltpu.async_copy(emb_hbm.at[idx_v.at[j]],
                             rows_v.at[pl.ds(j * _CH, _CH)], sem)
            for j in range(_NCH)
        ]
        for c in copies:
            c.wait()
        pltpu.sync_copy(rows_v, out_hbm.at[pl.ds(wid * _BPW, _BPW)])

    return _sc_gather


def kernel(x, W, b, emb):
    xr = x.reshape(N, C_IN, HW)
    br = b.reshape(D, 1)
    idx3, loss = _tc_call(xr, W, br, emb)
    rows = _sc_gather_call()(emb, idx3.reshape(_NW, _NCH, _CH))
    q = rows.reshape(N, HW, D).transpose(0, 2, 1).reshape(N, D, H, W_SP)
    return q, loss[0, 0]


# speedup vs baseline: 1.3976x; 1.0590x over previous
import functools

import jax
import jax.numpy as jnp
from jax import lax
from jax.experimental import pallas as pl
from jax.experimental.pallas import tpu as pltpu
from jax.experimental.pallas import tpu_sc as plsc

N, C_IN, H, W_SP = 16, 96, 32, 32
HW = H * W_SP
K, D = 8192, 32
KB = 2048
KBC = K // KB
_WIN = 4096 // KB
_LOSS_SCALE = 1.25 / float(N * D * HW)


def _vq_tc_body(x_ref, w_ref, b_ref, emb_ref, idx_ref, loss_ref,
                z_scr, z2_scr, minv_scr, arg_scr, acc_scr):
    n = pl.program_id(0)
    k = pl.program_id(1)

    @pl.when(k == 0)
    def _init():
        z = lax.dot_general(w_ref[...].astype(jnp.bfloat16),
                            x_ref[0].astype(jnp.bfloat16),
                            (((1,), (0,)), ((), ())),
                            preferred_element_type=jnp.float32)
        z = z + b_ref[...]
        z_scr[...] = (2.0 * z).astype(jnp.bfloat16)
        z2_scr[...] = jnp.sum(z * z, axis=0, keepdims=True)
        minv_scr[...] = jnp.full((1, HW), jnp.inf, jnp.float32)
        arg_scr[...] = jnp.zeros((1, HW), jnp.int32)

    @pl.when((n == 0) & (k == 0))
    def _init_acc():
        acc_scr[...] = jnp.zeros((1, 1), jnp.float32)

    emb = emb_ref[...]
    mneg = lax.dot_general((-emb).astype(jnp.bfloat16),
                           z_scr[...],
                           (((1,), (0,)), ((), ())),
                           preferred_element_type=jnp.float32)
    e2 = jnp.sum(emb * emb, axis=1, keepdims=True)
    score = (z2_scr[...] + mneg) + e2
    bmin = jnp.min(score, axis=0, keepdims=True)
    rows = lax.broadcasted_iota(jnp.int32, (KB, HW), 0)
    barg = jnp.min(jnp.where(score == bmin, rows, K),
                   axis=0, keepdims=True) + k * KB
    old = minv_scr[...]
    upd = bmin < old
    arg_scr[...] = jnp.where(upd, barg, arg_scr[...])
    newmin = jnp.where(upd, bmin, old)
    rounded = newmin.astype(jnp.bfloat16).astype(jnp.float32)
    minv_scr[...] = jnp.where((k % _WIN) == (_WIN - 1), rounded, newmin)

    @pl.when(k == KBC - 1)
    def _finish():
        idx_ref[0] = arg_scr[...]
        acc_scr[...] += jnp.sum(minv_scr[...], axis=1, keepdims=True)

        @pl.when(n == N - 1)
        def _emit_loss():
            loss_ref[...] = acc_scr[...] * _LOSS_SCALE


_tc_call = pl.pallas_call(
    _vq_tc_body,
    grid=(N, KBC),
    in_specs=[
        pl.BlockSpec((1, C_IN, HW), lambda n, k: (n, 0, 0)),
        pl.BlockSpec((D, C_IN), lambda n, k: (0, 0)),
        pl.BlockSpec((D, 1), lambda n, k: (0, 0)),
        pl.BlockSpec((KB, D), lambda n, k: (k, 0)),
    ],
    out_specs=[
        pl.BlockSpec((1, 1, HW), lambda n, k: (n, 0, 0)),
        pl.BlockSpec((1, 1), lambda n, k: (0, 0)),
    ],
    out_shape=[
        jax.ShapeDtypeStruct((N, 1, HW), jnp.int32),
        jax.ShapeDtypeStruct((1, 1), jnp.float32),
    ],
    scratch_shapes=[
        pltpu.VMEM((D, HW), jnp.bfloat16),
        pltpu.VMEM((1, HW), jnp.float32),
        pltpu.VMEM((1, HW), jnp.float32),
        pltpu.VMEM((1, HW), jnp.int32),
        pltpu.VMEM((1, 1), jnp.float32),
    ],
)

_NW = 32
_BPW = (N * HW) // _NW
_CH = 128
_NCH = _BPW // _CH


@functools.lru_cache(maxsize=1)
def _sc_gather_call():
    @functools.partial(
        pl.kernel,
        mesh=plsc.VectorSubcoreMesh(core_axis_name="c", subcore_axis_name="s"),
        out_type=jax.ShapeDtypeStruct((N * HW, D), jnp.float32),
        scratch_types=[
            pltpu.VMEM((_NCH, _CH), jnp.int32),
            pltpu.VMEM((_BPW, D), jnp.float32),
            pltpu.SemaphoreType.DMA,
        ],
        compiler_params=pltpu.CompilerParams(use_tc_tiling_on_sc=False),
    )
    def _sc_gather(emb_hbm, idx_hbm, out_hbm, idx_v, rows_v, sem):
        wid = lax.axis_index("s") * 2 + lax.axis_index("c")
        pltpu.sync_copy(idx_hbm.at[wid], idx_v)
        copies = [
            pltpu.async_copy(emb_hbm.at[idx_v.at[j]],
                             rows_v.at[pl.ds(j * _CH, _CH)], sem)
            for j in range(_NCH)
        ]
        for c in copies:
            c.wait()
        pltpu.sync_copy(rows_v, out_hbm.at[pl.ds(wid * _BPW, _BPW)])

    return _sc_gather


def kernel(x, W, b, emb):
    xr = x.reshape(N, C_IN, HW)
    br = b.reshape(D, 1)
    idx3, loss = _tc_call(xr, W, br, emb)
    rows = _sc_gather_call()(emb, idx3.reshape(_NW, _NCH, _CH))
    q = rows.reshape(N, HW, D).transpose(0, 2, 1).reshape(N, D, H, W_SP)
    return q, loss[0, 0]
